# baseline (speedup 1.0000x reference)
import jax
import jax.numpy as jnp
from jax.experimental import pallas as pl
from jax.experimental.pallas import tpu as pltpu

_E, _H, _D = 8, 768, 768
_T = 2048
_BT = 256
_NB = _T // _BT


def _main_body(elo_ref, x_ref, w_ref, o_ref):
    o_ref[...] = jnp.dot(
        x_ref[...], w_ref[0], preferred_element_type=jnp.float32
    )


def _fixup_body(sched_ref, grp_ref, o_in_ref, x_ref, w_ref, o_ref):
    i = pl.program_id(0)
    b = sched_ref[i, 0]
    e_lo = sched_ref[i, 1]
    e_hi = sched_ref[i, 2]
    xb = x_ref[...]
    row = jax.lax.broadcasted_iota(jnp.int32, (_BT, 1), 0) + b * _BT

    def body(e, acc):
        s = grp_ref[e, 0]
        t = grp_ref[e, 1]
        mask = (row >= s) & (row < t)
        prod = jnp.dot(xb, w_ref[e], preferred_element_type=jnp.float32)
        return jnp.where(mask, prod, acc)

    o_ref[...] = jax.lax.fori_loop(e_lo, e_hi + 1, body, o_in_ref[...])


def kernel(x, group_list, w):
    counts = group_list.astype(jnp.int32)
    ends = jnp.cumsum(counts)
    starts = ends - counts
    grp = jnp.stack([starts, ends], axis=1)

    blk_lo = jnp.arange(_NB, dtype=jnp.int32) * _BT
    e_lo = jnp.searchsorted(ends, blk_lo, side="right").astype(jnp.int32)
    e_hi = jnp.searchsorted(ends, blk_lo + (_BT - 1), side="right").astype(
        jnp.int32
    )
    e_hi = jnp.minimum(e_hi, _E - 1)

    main_spec = pltpu.PrefetchScalarGridSpec(
        num_scalar_prefetch=1,
        grid=(_NB,),
        in_specs=[
            pl.BlockSpec((_BT, _H), lambda i, elo: (i, 0)),
            pl.BlockSpec((1, _H, _D), lambda i, elo: (elo[i], 0, 0)),
        ],
        out_specs=pl.BlockSpec((_BT, _D), lambda i, elo: (i, 0)),
    )
    out = pl.pallas_call(
        _main_body,
        grid_spec=main_spec,
        out_shape=jax.ShapeDtypeStruct((_T, _D), jnp.float32),
    )(e_lo, x, w)

    straddled = e_hi > e_lo
    n_fix = jnp.sum(straddled.astype(jnp.int32))
    order = jnp.argsort(jnp.logical_not(straddled), stable=True).astype(
        jnp.int32
    )
    sched = jnp.stack(
        [order, e_lo[order], e_hi[order]], axis=1
    )

    fix_spec = pltpu.PrefetchScalarGridSpec(
        num_scalar_prefetch=2,
        grid=(n_fix,),
        in_specs=[
            pl.BlockSpec((_BT, _D), lambda i, sched, grp: (sched[i, 0], 0)),
            pl.BlockSpec((_BT, _H), lambda i, sched, grp: (sched[i, 0], 0)),
            pl.BlockSpec((_E, _H, _D), lambda i, sched, grp: (0, 0, 0)),
        ],
        out_specs=pl.BlockSpec((_BT, _D), lambda i, sched, grp: (sched[i, 0], 0)),
    )
    return pl.pallas_call(
        _fixup_body,
        grid_spec=fix_spec,
        out_shape=jax.ShapeDtypeStruct((_T, _D), jnp.float32),
        input_output_aliases={2: 0},
    )(sched, grp, out, x, w)


# speedup vs baseline: 1.0038x; 1.0038x over previous
import jax
import jax.numpy as jnp
from jax.experimental import pallas as pl
from jax.experimental.pallas import tpu as pltpu

_E, _H, _D = 8, 768, 768
_T = 2048
_BT = 256
_NB = _T // _BT
_MAX_STEPS = _NB + _E - 1


def _gmm_body(sched_ref, grp_ref, x_ref, w_ref, o_ref):
    i = pl.program_id(0)
    b = sched_ref[i, 0]
    e = sched_ref[i, 1]
    first = sched_ref[i, 2]
    s = grp_ref[e, 0]
    t = grp_ref[e, 1]
    base = b * _BT
    full = jnp.logical_and(s <= base, t >= base + _BT)

    def _masked_dot():
        row = jax.lax.broadcasted_iota(jnp.int32, (_BT, 1), 0) + base
        mask = (row >= s) & (row < t)
        xm = jnp.where(mask, x_ref[...], jnp.bfloat16(0))
        return jnp.dot(xm, w_ref[0], preferred_element_type=jnp.float32)

    @pl.when(jnp.logical_and(first == 1, full))
    def _():
        o_ref[...] = jnp.dot(
            x_ref[...], w_ref[0], preferred_element_type=jnp.float32
        )

    @pl.when(jnp.logical_and(first == 1, jnp.logical_not(full)))
    def _():
        o_ref[...] = _masked_dot()

    @pl.when(first == 0)
    def _():
        o_ref[...] += _masked_dot()


def kernel(x, group_list, w):
    counts = group_list.astype(jnp.int32)
    ends = jnp.cumsum(counts)
    starts = ends - counts
    grp = jnp.stack([starts, ends], axis=1)

    blk_lo = jnp.arange(_NB, dtype=jnp.int32) * _BT
    e_lo = jnp.searchsorted(ends, blk_lo, side="right").astype(jnp.int32)
    e_hi = jnp.searchsorted(ends, blk_lo + (_BT - 1), side="right").astype(
        jnp.int32
    )
    e_hi = jnp.minimum(e_hi, _E - 1)
    n_pairs = e_hi - e_lo + 1
    off = jnp.cumsum(n_pairs)
    total = off[-1]
    k = jnp.arange(_MAX_STEPS, dtype=jnp.int32)
    b_k = jnp.searchsorted(off, k, side="right").astype(jnp.int32)
    b_k = jnp.minimum(b_k, _NB - 1)
    pair_start = off[b_k] - n_pairs[b_k]
    e_k = e_lo[b_k] + (k - pair_start)
    first_k = (k == pair_start).astype(jnp.int32)
    sched = jnp.stack([b_k, e_k, first_k], axis=1)

    grid_spec = pltpu.PrefetchScalarGridSpec(
        num_scalar_prefetch=2,
        grid=(total,),
        in_specs=[
            pl.BlockSpec((_BT, _H), lambda i, sched, grp: (sched[i, 0], 0)),
            pl.BlockSpec((1, _H, _D), lambda i, sched, grp: (sched[i, 1], 0, 0)),
        ],
        out_specs=pl.BlockSpec((_BT, _D), lambda i, sched, grp: (sched[i, 0], 0)),
    )
    return pl.pallas_call(
        _gmm_body,
        grid_spec=grid_spec,
        out_shape=jax.ShapeDtypeStruct((_T, _D), jnp.float32),
    )(sched, grp, x, w)


# speedup vs baseline: 1.5856x; 1.5797x over previous
import jax
import jax.numpy as jnp
from jax.experimental import pallas as pl
from jax.experimental.pallas import tpu as pltpu

_E, _H, _D = 8, 768, 768
_T = 2048
_BT = 256
_NB = _T // _BT
_MAX_STEPS = _NB + _E - 1


def _fast_body(x_ref, w_ref, o_ref):
    o_ref[...] = jnp.dot(
        x_ref[...], w_ref[0], preferred_element_type=jnp.float32
    )


def _fast_path(x, counts, w):
    return pl.pallas_call(
        _fast_body,
        grid=(_NB,),
        in_specs=[
            pl.BlockSpec((_BT, _H), lambda i: (i, 0)),
            pl.BlockSpec((1, _H, _D), lambda i: (i, 0, 0)),
        ],
        out_specs=pl.BlockSpec((_BT, _D), lambda i: (i, 0)),
        out_shape=jax.ShapeDtypeStruct((_T, _D), jnp.float32),
    )(x, w)


def _gmm_body(sched_ref, grp_ref, x_ref, w_ref, o_ref):
    i = pl.program_id(0)
    b = sched_ref[i, 0]
    e = sched_ref[i, 1]
    first = sched_ref[i, 2]
    s = grp_ref[e, 0]
    t = grp_ref[e, 1]
    row = jax.lax.broadcasted_iota(jnp.int32, (_BT, 1), 0) + b * _BT
    mask = (row >= s) & (row < t)
    xm = jnp.where(mask, x_ref[...], jnp.bfloat16(0))
    acc = jnp.dot(xm, w_ref[0], preferred_element_type=jnp.float32)

    @pl.when(first == 1)
    def _():
        o_ref[...] = acc

    @pl.when(first == 0)
    def _():
        o_ref[...] += acc


def _general_path(x, counts, w):
    ends = jnp.cumsum(counts)
    starts = ends - counts
    grp = jnp.stack([starts, ends], axis=1)

    blk_lo = jnp.arange(_NB, dtype=jnp.int32) * _BT
    e_lo = jnp.searchsorted(ends, blk_lo, side="right").astype(jnp.int32)
    e_hi = jnp.searchsorted(ends, blk_lo + (_BT - 1), side="right").astype(
        jnp.int32
    )
    e_hi = jnp.minimum(e_hi, _E - 1)
    n_pairs = e_hi - e_lo + 1
    off = jnp.cumsum(n_pairs)
    total = off[-1]
    k = jnp.arange(_MAX_STEPS, dtype=jnp.int32)
    b_k = jnp.searchsorted(off, k, side="right").astype(jnp.int32)
    b_k = jnp.minimum(b_k, _NB - 1)
    pair_start = off[b_k] - n_pairs[b_k]
    e_k = e_lo[b_k] + (k - pair_start)
    first_k = (k == pair_start).astype(jnp.int32)
    sched = jnp.stack([b_k, e_k, first_k], axis=1)

    grid_spec = pltpu.PrefetchScalarGridSpec(
        num_scalar_prefetch=2,
        grid=(total,),
        in_specs=[
            pl.BlockSpec((_BT, _H), lambda i, sched, grp: (sched[i, 0], 0)),
            pl.BlockSpec((1, _H, _D), lambda i, sched, grp: (sched[i, 1], 0, 0)),
        ],
        out_specs=pl.BlockSpec((_BT, _D), lambda i, sched, grp: (sched[i, 0], 0)),
    )
    return pl.pallas_call(
        _gmm_body,
        grid_spec=grid_spec,
        out_shape=jax.ShapeDtypeStruct((_T, _D), jnp.float32),
    )(sched, grp, x, w)


def kernel(x, group_list, w):
    counts = group_list.astype(jnp.int32)
    uniform = jnp.all(counts == _T // _E)
    return jax.lax.cond(uniform, _fast_path, _general_path, x, counts, w)
